# baseline (speedup 1.0000x reference)
import functools

import jax
import jax.numpy as jnp
from jax import lax
from jax.experimental import pallas as pl
from jax.experimental.pallas import tpu as pltpu
from jax.experimental.pallas import tpu_sc as plsc

D_MODEL = 128
SEQ = 200
G = 80
NUM_WORKERS = 32
LANES = 16
PE_EXT = SEQ + G - 40
GB = 4
OB = 2
PERIOD = 8
HBM_SET = (1, 4, 6)
TBL_PAD = 1024


def _build_kernel(n_tokens):
    per_w = n_tokens // NUM_WORKERS
    nchunks = per_w // G
    assert nchunks % PERIOD == 0 and nchunks >= 2 * PERIOD

    mesh = plsc.VectorSubcoreMesh(core_axis_name="c", subcore_axis_name="s")

    @functools.partial(
        pl.kernel,
        out_type=jax.ShapeDtypeStruct((n_tokens, D_MODEL), jnp.float32),
        mesh=mesh,
        scratch_types=[
            pltpu.VMEM((per_w,), jnp.int32),
            pltpu.VMEM((PE_EXT, D_MODEL), jnp.float32),
            pltpu.VMEM_SHARED((TBL_PAD, D_MODEL), jnp.float32),
        ]
        + [pltpu.VMEM((G, D_MODEL), jnp.float32)] * (GB + OB)
        + [pltpu.SemaphoreType.DMA] * (GB + OB),
    )
    def embed(tok_hbm, table_hbm, pe_hbm, out_hbm, tok_v, pe_v, table_s, *bufs):
        gs = bufs[:GB]
        os_ = bufs[GB:GB + OB]
        sgs = bufs[GB + OB:2 * GB + OB]
        sws = bufs[2 * GB + OB:]

        sub = lax.axis_index("s")
        wid = sub * 2 + lax.axis_index("c")
        base = wid * per_w

        rows_per_sub = TBL_PAD // 16
        pltpu.sync_copy(table_hbm.at[pl.ds(sub * rows_per_sub, rows_per_sub)],
                        table_s.at[pl.ds(sub * rows_per_sub, rows_per_sub)])
        pltpu.sync_copy(tok_hbm.at[pl.ds(base, per_w)], tok_v)
        pltpu.sync_copy(pe_hbm, pe_v.at[pl.ds(0, SEQ)])
        pltpu.sync_copy(pe_hbm.at[pl.ds(0, PE_EXT - SEQ)],
                        pe_v.at[pl.ds(SEQ, PE_EXT - SEQ)])
        plsc.subcore_barrier()

        def src_tbl(m):
            return table_hbm if (m % PERIOD) in HBM_SET else table_s

        def start_gather(c, s, m):
            pltpu.async_copy(
                src_tbl(m).at[tok_v.at[pl.ds(c * G, G)]], gs[s], sgs[s])

        def wait_gather(s, m):
            pltpu.make_async_copy(
                src_tbl(m).at[tok_v.at[pl.ds(0, G)]], gs[s], sgs[s]).wait()

        def start_write(c, s):
            pltpu.async_copy(
                os_[s], out_hbm.at[pl.ds(base + c * G, G)], sws[s])

        def wait_write(s):
            pltpu.make_async_copy(
                os_[s], out_hbm.at[pl.ds(base, G)], sws[s]).wait()

        def compute(c, s, so):
            pb = (c * G) % SEQ
            gv = gs[s]
            ov = os_[so]

            @plsc.parallel_loop(0, G, unroll=2)
            def _row(t):
                pr = pb + t
                slices = [pl.ds(j * LANES, LANES) for j in range(D_MODEL // LANES)]
                gvals = [gv[t, sl] for sl in slices]
                pvals = [pe_v[pr, sl] for sl in slices]
                for sl, gval, pval in zip(slices, gvals, pvals):
                    ov[t, sl] = gval + pval

        for c in range(GB):
            start_gather(c, c, c)
        for c in range(GB):
            wait_gather(c, c)
            if c >= OB:
                wait_write(c % OB)
            compute(c, c, c % OB)
            start_write(c, c % OB)
            start_gather(c + GB, c, c + GB)

        @pl.loop(GB, nchunks - GB, step=PERIOD)
        def _steady(c0):
            for j in range(PERIOD):
                c = c0 + j
                k = j % GB
                m = GB + j
                wait_gather(k, m)
                wait_write(k % OB)
                compute(c, k, k % OB)
                start_write(c, k % OB)
                start_gather(c + GB, k, m + GB)

        for k in range(GB):
            c = nchunks - GB + k
            wait_gather(k, GB + k)
            wait_write(k % OB)
            compute(c, k, k % OB)
            start_write(c, k % OB)
        for s in range(OB):
            wait_write(s)

    return embed


def kernel(tokens, table, PE):
    batch, seq = tokens.shape
    n_tokens = batch * seq
    vocab = table.shape[0]
    table_p = jnp.concatenate(
        [table, jnp.zeros((TBL_PAD - vocab, table.shape[1]), table.dtype)])
    out = _build_kernel(n_tokens)(
        tokens.reshape(n_tokens), table_p, PE[:seq])
    return out.reshape(batch, seq, D_MODEL)


# speedup vs baseline: 1.0454x; 1.0454x over previous
import functools

import jax
import jax.numpy as jnp
from jax import lax
from jax.experimental import pallas as pl
from jax.experimental.pallas import tpu as pltpu
from jax.experimental.pallas import tpu_sc as plsc

D_MODEL = 128
SEQ = 200
G = 80
NUM_WORKERS = 32
LANES = 16
PE_EXT = SEQ + G - 40
NB = 5
K = 3
TBL_PAD = 1024


def _build_kernel(n_tokens):
    per_w = n_tokens // NUM_WORKERS
    nchunks = per_w // G
    assert nchunks % NB == 0 and nchunks >= 2 * NB

    mesh = plsc.VectorSubcoreMesh(core_axis_name="c", subcore_axis_name="s")

    @functools.partial(
        pl.kernel,
        out_type=jax.ShapeDtypeStruct((n_tokens, D_MODEL), jnp.float32),
        mesh=mesh,
        scratch_types=[
            pltpu.VMEM((per_w,), jnp.int32),
            pltpu.VMEM((PE_EXT, D_MODEL), jnp.float32),
            pltpu.VMEM_SHARED((TBL_PAD, D_MODEL), jnp.float32),
        ]
        + [pltpu.VMEM((G, D_MODEL), jnp.float32)] * NB
        + [pltpu.SemaphoreType.DMA] * (2 * NB),
    )
    def embed(tok_hbm, table_hbm, pe_hbm, out_hbm, tok_v, pe_v, table_s, *bufs):
        bs = bufs[:NB]
        sgs = bufs[NB:2 * NB]
        sws = bufs[2 * NB:]

        sub = lax.axis_index("s")
        wid = sub * 2 + lax.axis_index("c")
        base = wid * per_w

        rows_per_sub = TBL_PAD // 16
        pltpu.sync_copy(table_hbm.at[pl.ds(sub * rows_per_sub, rows_per_sub)],
                        table_s.at[pl.ds(sub * rows_per_sub, rows_per_sub)])
        pltpu.sync_copy(tok_hbm.at[pl.ds(base, per_w)], tok_v)
        pltpu.sync_copy(pe_hbm, pe_v.at[pl.ds(0, SEQ)])
        pltpu.sync_copy(pe_hbm.at[pl.ds(0, PE_EXT - SEQ)],
                        pe_v.at[pl.ds(SEQ, PE_EXT - SEQ)])
        plsc.subcore_barrier()

        def start_gather(c, b):
            pltpu.async_copy(
                table_s.at[tok_v.at[pl.ds(c * G, G)]], bs[b], sgs[b])

        def wait_gather(b):
            pltpu.make_async_copy(
                table_s.at[tok_v.at[pl.ds(0, G)]], bs[b], sgs[b]).wait()

        def start_write(c, b):
            pltpu.async_copy(
                bs[b], out_hbm.at[pl.ds(base + c * G, G)], sws[b])

        def wait_write(b):
            pltpu.make_async_copy(
                bs[b], out_hbm.at[pl.ds(base, G)], sws[b]).wait()

        def compute(c, b):
            pb = (c * G) % SEQ
            bv = bs[b]

            @plsc.parallel_loop(0, G, unroll=2)
            def _row(t):
                pr = pb + t
                slices = [pl.ds(j * LANES, LANES) for j in range(D_MODEL // LANES)]
                pvals = [pe_v[pr, sl] for sl in slices]
                for sl, pval in zip(slices, pvals):
                    plsc.addupdate(bv.at[t, sl], pval)

        for c in range(K):
            start_gather(c, c % NB)
        for c in range(NB):
            wait_gather(c % NB)
            compute(c, c % NB)
            start_write(c, c % NB)
            if c >= 2:
                wait_write((c - 2) % NB)
            start_gather(c + K, (c + K) % NB)

        @pl.loop(NB, nchunks - NB, step=NB)
        def _steady(c0):
            for j in range(NB):
                c = c0 + j
                wait_gather(j)
                compute(c, j)
                start_write(c, j)
                wait_write((j - 2) % NB)
                start_gather(c + K, (j + K) % NB)

        for j in range(NB):
            c = nchunks - NB + j
            wait_gather(j)
            compute(c, j)
            start_write(c, j)
            if j < NB - K:
                wait_write((j - 2) % NB)
                start_gather(c + K, (j + K) % NB)
        for b in range(NB):
            wait_write(b)

    return embed


def kernel(tokens, table, PE):
    batch, seq = tokens.shape
    n_tokens = batch * seq
    vocab = table.shape[0]
    table_p = jnp.concatenate(
        [table, jnp.zeros((TBL_PAD - vocab, table.shape[1]), table.dtype)])
    out = _build_kernel(n_tokens)(
        tokens.reshape(n_tokens), table_p, PE[:seq])
    return out.reshape(batch, seq, D_MODEL)


# speedup vs baseline: 1.2653x; 1.2103x over previous
import functools

import jax
import jax.numpy as jnp
from jax import lax
from jax.experimental import pallas as pl
from jax.experimental.pallas import tpu as pltpu
from jax.experimental.pallas import tpu_sc as plsc

D_MODEL = 128
SEQ = 200
BATCH = 4096
G = 128
NUM_WORKERS = 32
LANES = 16
PE_ROWS = 16
NB = 5
K = 3
TBL_PAD = 1024


def _build_kernel(n_tokens):
    per_w = n_tokens // NUM_WORKERS
    nchunks = per_w // G
    assert nchunks % NB == 0 and nchunks >= 2 * NB

    mesh = plsc.VectorSubcoreMesh(core_axis_name="c", subcore_axis_name="s")

    @functools.partial(
        pl.kernel,
        out_type=jax.ShapeDtypeStruct((n_tokens, D_MODEL), jnp.float32),
        mesh=mesh,
        scratch_types=[
            pltpu.VMEM((per_w,), jnp.int32),
            pltpu.VMEM((PE_ROWS, D_MODEL), jnp.float32),
            pltpu.VMEM_SHARED((TBL_PAD, D_MODEL), jnp.float32),
        ]
        + [pltpu.VMEM((G, D_MODEL), jnp.float32)] * NB
        + [pltpu.VMEM((G,), jnp.int32)] * NB
        + [pltpu.SemaphoreType.DMA] * (2 * NB),
    )
    def embed(tok_hbm, table_hbm, pe_hbm, out_hbm, tok_v, pe_v, table_s, *bufs):
        bs = bufs[:NB]
        ixs = bufs[NB:2 * NB]
        sgs = bufs[2 * NB:3 * NB]
        sws = bufs[3 * NB:]

        sub = lax.axis_index("s")
        wid = sub * 2 + lax.axis_index("c")
        base = wid * per_w
        l0 = pl.multiple_of((base >> 12) & ~7, 8)

        rows_per_sub = TBL_PAD // 16
        pltpu.sync_copy(table_hbm.at[pl.ds(sub * rows_per_sub, rows_per_sub)],
                        table_s.at[pl.ds(sub * rows_per_sub, rows_per_sub)])
        pltpu.sync_copy(tok_hbm.at[pl.ds(base, per_w)], tok_v)
        pltpu.sync_copy(pe_hbm.at[pl.ds(l0, PE_ROWS)], pe_v)
        plsc.subcore_barrier()

        slices = [pl.ds(j * LANES, LANES) for j in range(D_MODEL // LANES)]

        def start_gather(c, b):
            pltpu.async_copy(
                table_s.at[tok_v.at[pl.ds(c * G, G)]], bs[b], sgs[b])

        def wait_gather(b):
            pltpu.make_async_copy(
                table_s.at[tok_v.at[pl.ds(0, G)]], bs[b], sgs[b]).wait()

        def start_write(b):
            pltpu.async_copy(bs[b], out_hbm.at[ixs[b]], sws[b])

        def wait_write(b):
            pltpu.make_async_copy(bs[b], out_hbm.at[ixs[b]], sws[b]).wait()

        def compute(c, b):
            p0 = base + c * G
            l = p0 >> 12
            b0 = p0 & (BATCH - 1)
            bv = bs[b]
            iv = ixs[b]

            pvals = [pe_v[l - l0, sl] for sl in slices]

            lane = lax.iota(jnp.int32, LANES) * SEQ
            row0 = b0 * SEQ + l
            for j in range(G // LANES):
                iv[pl.ds(j * LANES, LANES)] = lane + (row0 + j * LANES * SEQ)

            @plsc.parallel_loop(0, G, unroll=2)
            def _row(t):
                for sl, pval in zip(slices, pvals):
                    plsc.addupdate(bv.at[t, sl], pval)

        for c in range(K):
            start_gather(c, c % NB)
        for c in range(NB):
            wait_gather(c % NB)
            compute(c, c % NB)
            start_write(c % NB)
            if c >= 2:
                wait_write((c - 2) % NB)
            start_gather(c + K, (c + K) % NB)

        @pl.loop(NB, nchunks - NB, step=NB)
        def _steady(c0):
            for j in range(NB):
                c = c0 + j
                wait_gather(j)
                compute(c, j)
                start_write(j)
                wait_write((j - 2) % NB)
                start_gather(c + K, (j + K) % NB)

        for j in range(NB):
            c = nchunks - NB + j
            wait_gather(j)
            compute(c, j)
            start_write(j)
            if j < NB - K:
                wait_write((j - 2) % NB)
                start_gather(c + K, (j + K) % NB)
        for b in range(NB):
            wait_write(b)

    return embed


def kernel(tokens, table, PE):
    batch, seq = tokens.shape
    n_tokens = batch * seq
    vocab = table.shape[0]
    table_p = jnp.concatenate(
        [table, jnp.zeros((TBL_PAD - vocab, table.shape[1]), table.dtype)])
    tok_t = tokens.T.reshape(n_tokens)
    out = _build_kernel(n_tokens)(tok_t, table_p, PE)
    return out.reshape(batch, seq, D_MODEL)


# speedup vs baseline: 1.2661x; 1.0006x over previous
import functools

import jax
import jax.numpy as jnp
from jax import lax
from jax.experimental import pallas as pl
from jax.experimental.pallas import tpu as pltpu
from jax.experimental.pallas import tpu_sc as plsc

D_MODEL = 128
SEQ = 200
BATCH = 4096
G = 128
NUM_WORKERS = 32
LANES = 16
PE_ROWS = 16
NB = 5
K = 3
TBL_PAD = 1024


def _build_kernel(n_tokens):
    per_w = n_tokens // NUM_WORKERS
    nchunks = per_w // G
    assert nchunks % NB == 0 and nchunks >= 2 * NB

    mesh = plsc.VectorSubcoreMesh(core_axis_name="c", subcore_axis_name="s")

    @functools.partial(
        pl.kernel,
        out_type=jax.ShapeDtypeStruct((n_tokens, D_MODEL), jnp.float32),
        mesh=mesh,
        scratch_types=[
            pltpu.VMEM((per_w,), jnp.int32),
            pltpu.VMEM((PE_ROWS, D_MODEL), jnp.float32),
            pltpu.VMEM_SHARED((TBL_PAD, D_MODEL), jnp.float32),
        ]
        + [pltpu.VMEM((G, D_MODEL), jnp.float32)] * NB
        + [pltpu.VMEM((G,), jnp.int32)] * NB
        + [pltpu.SemaphoreType.DMA] * (2 * NB),
    )
    def embed(tok_hbm, table_hbm, pe_hbm, out_hbm, tok_v, pe_v, table_s, *bufs):
        bs = bufs[:NB]
        ixs = bufs[NB:2 * NB]
        sgs = bufs[2 * NB:3 * NB]
        sws = bufs[3 * NB:]

        sub = lax.axis_index("s")
        wid = sub * 2 + lax.axis_index("c")
        base = wid * per_w
        l0 = pl.multiple_of((base >> 12) & ~7, 8)

        rows_per_sub = TBL_PAD // 16
        pltpu.sync_copy(table_hbm.at[pl.ds(sub * rows_per_sub, rows_per_sub)],
                        table_s.at[pl.ds(sub * rows_per_sub, rows_per_sub)])
        pltpu.sync_copy(tok_hbm.at[pl.ds(base, per_w)], tok_v)
        pltpu.sync_copy(pe_hbm.at[pl.ds(l0, PE_ROWS)], pe_v)
        plsc.subcore_barrier()

        slices = [pl.ds(j * LANES, LANES) for j in range(D_MODEL // LANES)]

        def start_gather(c, b):
            pltpu.async_copy(
                table_s.at[tok_v.at[pl.ds(c * G, G)]], bs[b], sgs[b])

        def wait_gather(b):
            pltpu.make_async_copy(
                table_s.at[tok_v.at[pl.ds(0, G)]], bs[b], sgs[b]).wait()

        def start_write(b):
            pltpu.async_copy(bs[b], out_hbm.at[ixs[b]], sws[b])

        def wait_write(b):
            pltpu.make_async_copy(bs[b], out_hbm.at[ixs[b]], sws[b]).wait()

        def prelude(c, b):
            p0 = base + c * G
            l = p0 >> 12
            b0 = p0 & (BATCH - 1)
            iv = ixs[b]
            pvals = [pe_v[l - l0, sl] for sl in slices]
            lane = lax.iota(jnp.int32, LANES) * SEQ
            row0 = b0 * SEQ + l
            for j in range(G // LANES):
                iv[pl.ds(j * LANES, LANES)] = lane + (row0 + j * LANES * SEQ)
            return pvals

        def compute(b, pvals):
            bv = bs[b]

            @plsc.parallel_loop(0, G, unroll=4)
            def _row(t):
                for sl, pval in zip(slices, pvals):
                    plsc.addupdate(bv.at[t, sl], pval)

        for c in range(K):
            start_gather(c, c % NB)
        for c in range(NB):
            pvals = prelude(c, c % NB)
            wait_gather(c % NB)
            compute(c % NB, pvals)
            start_write(c % NB)
            if c >= 2:
                wait_write((c - 2) % NB)
            start_gather(c + K, (c + K) % NB)

        @pl.loop(NB, nchunks - NB, step=NB)
        def _steady(c0):
            for j in range(NB):
                c = c0 + j
                pvals = prelude(c, j)
                wait_gather(j)
                compute(j, pvals)
                start_write(j)
                wait_write((j - 2) % NB)
                start_gather(c + K, (j + K) % NB)

        for j in range(NB):
            c = nchunks - NB + j
            pvals = prelude(c, j)
            wait_gather(j)
            compute(j, pvals)
            start_write(j)
            if j < NB - K:
                wait_write((j - 2) % NB)
                start_gather(c + K, (j + K) % NB)
        for b in range(NB):
            wait_write(b)

    return embed


def kernel(tokens, table, PE):
    batch, seq = tokens.shape
    n_tokens = batch * seq
    vocab = table.shape[0]
    table_p = jnp.concatenate(
        [table, jnp.zeros((TBL_PAD - vocab, table.shape[1]), table.dtype)])
    tok_t = tokens.T.reshape(n_tokens)
    out = _build_kernel(n_tokens)(tok_t, table_p, PE)
    return out.reshape(batch, seq, D_MODEL)
